# baseline (speedup 1.0000x reference)
import dataclasses
import functools

import jax
import jax.numpy as jnp
from jax.experimental import pallas as pl
from jax.experimental.pallas import tpu as pltpu
from jax.experimental.pallas import tpu_sc as plsc

TOKENS = 2048
D_IN = 1024
N_EXPERTS = 8
D_HID = 1024
D_OUT = 1024
TOP_K = 2

TILE_M = 128
M_PAD = TOKENS * TOP_K + N_EXPERTS * TILE_M
N_TILES = M_PAD // TILE_M

SC_LANES = 16
SC_ROW = 128
ROW_SPLIT = D_IN // SC_ROW
SC_W = 128

@functools.cache
def _vector_mesh():
    return plsc.VectorSubcoreMesh(core_axis_name="c", subcore_axis_name="s")


def _sc_compiler_params():
    cp = pltpu.CompilerParams()
    if "needs_layout_passes" in pltpu.CompilerParams.__dataclass_fields__:
        cp = dataclasses.replace(cp, needs_layout_passes=False)
    return cp


def _gate_body(x_ref, gw_ref, gb_ref, idx_ref, wts_ref):
    logits = jnp.dot(x_ref[...], gw_ref[...],
                     preferred_element_type=jnp.float32) + gb_ref[...]
    col = jax.lax.broadcasted_iota(jnp.int32, logits.shape, 1)
    m1 = jnp.max(logits, axis=1, keepdims=True)
    i1 = jnp.min(jnp.where(logits == m1, col, N_EXPERTS), axis=1,
                 keepdims=True)
    l2 = jnp.where(col == i1, -jnp.inf, logits)
    m2 = jnp.max(l2, axis=1, keepdims=True)
    i2 = jnp.min(jnp.where(l2 == m2, col, N_EXPERTS), axis=1,
                 keepdims=True)
    b = jnp.exp(m2 - m1)
    wa = 1.0 / (1.0 + b)
    wb = b / (1.0 + b)
    idx_ref[...] = jnp.concatenate([i1, i2], axis=1)
    wts_ref[...] = jnp.concatenate([wa, wb], axis=1)


def _gate(x, gate_W, gb2d):
    return pl.pallas_call(
        _gate_body,
        grid=(1,),
        in_specs=[
            pl.BlockSpec((TOKENS, D_IN), lambda i: (0, 0)),
            pl.BlockSpec((D_IN, N_EXPERTS), lambda i: (0, 0)),
            pl.BlockSpec((1, N_EXPERTS), lambda i: (0, 0)),
        ],
        out_specs=[
            pl.BlockSpec((TOKENS, TOP_K), lambda i: (0, 0)),
            pl.BlockSpec((TOKENS, TOP_K), lambda i: (0, 0)),
        ],
        out_shape=[
            jax.ShapeDtypeStruct((TOKENS, TOP_K), jnp.int32),
            jax.ShapeDtypeStruct((TOKENS, TOP_K), jnp.float32),
        ],
    )(x, gate_W, gb2d)


def _dispatch(x_r, pos0e, pos1e, wts0B, wts1B, pos0t, pos1t):
    n_rows = TOKENS * ROW_SPLIT

    @pl.kernel(
        out_type=[
            jax.ShapeDtypeStruct((M_PAD * ROW_SPLIT, SC_ROW), jnp.float32),
            jax.ShapeDtypeStruct((M_PAD, SC_ROW), jnp.float32),
        ],
        mesh=_vector_mesh(),
        compiler_params=_sc_compiler_params(),
    )
    def k(x_hbm, p0_hbm, p1_hbm, w0_hbm, w1_hbm, t0_hbm, t1_hbm,
          xs_hbm, wpad_hbm):
        def scatter_rows(v_hbm, p_hbm, o_hbm, n):
            def sbody(x_v, i_v):
                pltpu.sync_copy(x_v, o_hbm.at[i_v.at[0]])
            pltpu.emit_pipeline(
                sbody,
                grid=(n // SC_W,),
                in_specs=[
                    pl.BlockSpec((SC_W, SC_ROW), lambda i: (i, 0)),
                    pl.BlockSpec((1, SC_W), lambda i: (0, i)),
                ],
                out_specs=[],
                core_axis_name=("c", "s"),
                dimension_semantics=(pltpu.PARALLEL,),
            )(v_hbm, p_hbm)

        scatter_rows(x_hbm, p0_hbm, xs_hbm, n_rows)
        scatter_rows(x_hbm, p1_hbm, xs_hbm, n_rows)
        scatter_rows(w0_hbm, t0_hbm, wpad_hbm, TOKENS)
        scatter_rows(w1_hbm, t1_hbm, wpad_hbm, TOKENS)

    return k(x_r, pos0e, pos1e, wts0B, wts1B, pos0t, pos1t)


def _gmm_body(te_ref, xs_ref, wp_ref, w1_ref, b1_ref, w2_ref, b2_ref, y_ref):
    h = jnp.maximum(
        jnp.dot(xs_ref[...].astype(jnp.bfloat16), w1_ref[0],
                preferred_element_type=jnp.float32) + b1_ref[0], 0.0)
    y = jnp.dot(h.astype(jnp.bfloat16), w2_ref[0],
                preferred_element_type=jnp.float32) + b2_ref[0]
    y_ref[...] = y * wp_ref[:, 0:1]


def _gmm(tile_expert, xs, wpad, w1bf, b1r, w2bf, b2r):
    grid_spec = pltpu.PrefetchScalarGridSpec(
        num_scalar_prefetch=1,
        grid=(N_TILES,),
        in_specs=[
            pl.BlockSpec((TILE_M, D_IN), lambda i, s: (i, 0)),
            pl.BlockSpec((TILE_M, SC_ROW), lambda i, s: (i, 0)),
            pl.BlockSpec((1, D_IN, D_HID), lambda i, s: (s[i], 0, 0)),
            pl.BlockSpec((1, 1, D_HID), lambda i, s: (s[i], 0, 0)),
            pl.BlockSpec((1, D_HID, D_OUT), lambda i, s: (s[i], 0, 0)),
            pl.BlockSpec((1, 1, D_OUT), lambda i, s: (s[i], 0, 0)),
        ],
        out_specs=pl.BlockSpec((TILE_M, D_OUT), lambda i, s: (i, 0)),
    )
    return pl.pallas_call(
        _gmm_body,
        grid_spec=grid_spec,
        out_shape=jax.ShapeDtypeStruct((M_PAD, D_OUT), jnp.float32),
    )(tile_expert, xs, wpad, w1bf, b1r, w2bf, b2r)


def _combine(y_r, pos0e, pos1e):
    n_rows = TOKENS * ROW_SPLIT

    @pl.kernel(
        out_type=jax.ShapeDtypeStruct((n_rows, SC_ROW), jnp.float32),
        mesh=_vector_mesh(),
        scratch_types=[
            pltpu.VMEM((SC_W, SC_ROW), jnp.float32),
        ],
        compiler_params=_sc_compiler_params(),
    )
    def k(y_hbm, p0_hbm, p1_hbm, o_hbm, bufb):
        def cbody(i0_v, i1_v, o_v):
            pltpu.sync_copy(y_hbm.at[i0_v.at[0]], o_v)
            pltpu.sync_copy(y_hbm.at[i1_v.at[0]], bufb)

            @pl.loop(0, SC_W)
            def _(r):
                @pl.loop(0, SC_ROW, step=SC_LANES)
                def _(c):
                    slc = (r, pl.ds(c, SC_LANES))
                    o_v.at[*slc][...] = (
                        o_v.at[*slc][...] + bufb.at[*slc][...])

        pltpu.emit_pipeline(
            cbody,
            grid=(n_rows // SC_W,),
            in_specs=[
                pl.BlockSpec((1, SC_W), lambda i: (0, i)),
                pl.BlockSpec((1, SC_W), lambda i: (0, i)),
            ],
            out_specs=[
                pl.BlockSpec((SC_W, SC_ROW), lambda i: (i, 0)),
            ],
            core_axis_name=("c", "s"),
            dimension_semantics=(pltpu.PARALLEL,),
        )(p0_hbm, p1_hbm, o_hbm)

    return k(y_r, pos0e, pos1e)


@jax.jit
def kernel(x, gate_W, gate_b, W1, b1, W2, b2):
    gb2d = gate_b.reshape(1, N_EXPERTS)
    b1r = b1.reshape(N_EXPERTS, 1, D_HID)
    b2r = b2.reshape(N_EXPERTS, 1, D_OUT)
    w1bf = W1.astype(jnp.bfloat16)
    w2bf = W2.astype(jnp.bfloat16)
    idx, wts = _gate(x, gate_W, gb2d)

    entries = idx.reshape(TOKENS * TOP_K)
    oh = (entries[:, None] == jnp.arange(N_EXPERTS)[None, :]).astype(jnp.int32)
    cum = jnp.cumsum(oh, axis=0)
    counts = cum[-1]
    padded = ((counts + TILE_M - 1) // TILE_M) * TILE_M
    cum_end = jnp.cumsum(padded)
    off = cum_end - padded
    pos = jnp.sum((cum - 1 + off[None, :]) * oh, axis=1).astype(jnp.int32)
    pos2 = pos.reshape(TOKENS, TOP_K)
    tile_start = jnp.arange(N_TILES, dtype=jnp.int32) * TILE_M
    tile_expert = jnp.minimum(
        jnp.sum((tile_start[:, None] >= cum_end[None, :]).astype(jnp.int32),
                axis=1),
        N_EXPERTS - 1).astype(jnp.int32)

    sub = jnp.arange(ROW_SPLIT, dtype=jnp.int32)[None, :]
    pos0e = (pos2[:, 0:1] * ROW_SPLIT + sub).reshape(1, TOKENS * ROW_SPLIT)
    pos1e = (pos2[:, 1:2] * ROW_SPLIT + sub).reshape(1, TOKENS * ROW_SPLIT)
    pos0t = pos2[:, 0].reshape(1, TOKENS)
    pos1t = pos2[:, 1].reshape(1, TOKENS)
    wts0B = jnp.broadcast_to(wts[:, 0:1], (TOKENS, SC_ROW))
    wts1B = jnp.broadcast_to(wts[:, 1:2], (TOKENS, SC_ROW))

    x_r = x.reshape(TOKENS * ROW_SPLIT, SC_ROW)
    xs_r, wpad = _dispatch(x_r, pos0e, pos1e, wts0B, wts1B, pos0t, pos1t)
    xs = xs_r.reshape(M_PAD, D_IN)
    y = _gmm(tile_expert, xs, wpad, w1bf, b1r, w2bf, b2r)
    y_r = y.reshape(M_PAD * ROW_SPLIT, SC_ROW)
    out_r = _combine(y_r, pos0e, pos1e)
    return out_r.reshape(TOKENS, D_OUT)


# speedup vs baseline: 1.7616x; 1.7616x over previous
import jax
import jax.numpy as jnp
from jax.experimental import pallas as pl
from jax.experimental.pallas import tpu as pltpu

TOKENS = 2048
D_IN = 1024
N_EXPERTS = 8
D_HID = 1024
D_OUT = 1024
TOP_K = 2
TILE_M = 512


def _moe_kernel(x_ref, xbf_ref, gw_ref, gb_ref, w1_ref, b1_ref, w2_ref,
                b2_ref, out_ref, w_scr):
    e = pl.program_id(0)

    @pl.when(e == 0)
    def _gate():
        logits = jnp.dot(x_ref[...], gw_ref[...],
                         preferred_element_type=jnp.float32) + gb_ref[...]
        col = jax.lax.broadcasted_iota(jnp.int32, logits.shape, 1)
        m1 = jnp.max(logits, axis=1, keepdims=True)
        i1 = jnp.min(jnp.where(logits == m1, col, N_EXPERTS), axis=1,
                     keepdims=True)
        l2 = jnp.where(col == i1, -jnp.inf, logits)
        m2 = jnp.max(l2, axis=1, keepdims=True)
        i2 = jnp.min(jnp.where(l2 == m2, col, N_EXPERTS), axis=1,
                     keepdims=True)
        b = jnp.exp(m2 - m1)
        wa = 1.0 / (1.0 + b)
        wb = b / (1.0 + b)
        w_scr[...] = jnp.where(col == i1, wa,
                               jnp.where(col == i2, wb, 0.0))

    w1 = w1_ref[0]
    w2 = w2_ref[0]
    b1v = b1_ref[0]
    b2v = b2_ref[0]

    def body(i, _):
        sl = pl.ds(i * TILE_M, TILE_M)
        h = jnp.maximum(
            jnp.dot(xbf_ref[sl, :], w1,
                    preferred_element_type=jnp.float32) + b1v, 0.0)
        y = jnp.dot(h.astype(jnp.bfloat16), w2,
                    preferred_element_type=jnp.float32) + b2v
        wt = w_scr[sl, :]
        col = jax.lax.broadcasted_iota(jnp.int32, wt.shape, 1)
        we = jnp.sum(jnp.where(col == e, wt, 0.0), axis=1, keepdims=True)
        contrib = we * y

        @pl.when(e == 0)
        def _init():
            out_ref[sl, :] = contrib

        @pl.when(e > 0)
        def _acc():
            out_ref[sl, :] = out_ref[sl, :] + contrib

        return 0

    jax.lax.fori_loop(0, TOKENS // TILE_M, body, 0)


@jax.jit
def kernel(x, gate_W, gate_b, W1, b1, W2, b2):
    gb2d = gate_b.reshape(1, N_EXPERTS)
    b1r = b1.reshape(N_EXPERTS, 1, D_HID)
    b2r = b2.reshape(N_EXPERTS, 1, D_OUT)
    w1bf = W1.astype(jnp.bfloat16)
    w2bf = W2.astype(jnp.bfloat16)
    xbf = x.astype(jnp.bfloat16)
    return pl.pallas_call(
        _moe_kernel,
        grid=(N_EXPERTS,),
        in_specs=[
            pl.BlockSpec((TOKENS, D_IN), lambda e: (0, 0)),
            pl.BlockSpec((TOKENS, D_IN), lambda e: (0, 0)),
            pl.BlockSpec((D_IN, N_EXPERTS), lambda e: (0, 0)),
            pl.BlockSpec((1, N_EXPERTS), lambda e: (0, 0)),
            pl.BlockSpec((1, D_IN, D_HID), lambda e: (e, 0, 0)),
            pl.BlockSpec((1, 1, D_HID), lambda e: (e, 0, 0)),
            pl.BlockSpec((1, D_HID, D_OUT), lambda e: (e, 0, 0)),
            pl.BlockSpec((1, 1, D_OUT), lambda e: (e, 0, 0)),
        ],
        out_specs=pl.BlockSpec((TOKENS, D_OUT), lambda e: (0, 0)),
        out_shape=jax.ShapeDtypeStruct((TOKENS, D_OUT), jnp.float32),
        scratch_shapes=[pltpu.VMEM((TOKENS, N_EXPERTS), jnp.float32)],
    )(x, xbf, gate_W, gb2d, w1bf, b1r, w2bf, b2r)


# speedup vs baseline: 2.3445x; 1.3309x over previous
import jax
import jax.numpy as jnp
from jax.experimental import pallas as pl
from jax.experimental.pallas import tpu as pltpu

TOKENS = 2048
D_IN = 1024
N_EXPERTS = 8
D_HID = 1024
D_OUT = 1024
TOP_K = 2
TILE_M = 512


def _moe_kernel(x_ref, gw_ref, gb_ref, w1_ref, b1_ref, w2_ref,
                b2_ref, out_ref, w_scr):
    e = pl.program_id(0)

    @pl.when(e == 0)
    def _gate():
        logits = jnp.dot(x_ref[...], gw_ref[...],
                         preferred_element_type=jnp.float32) + gb_ref[...]
        col = jax.lax.broadcasted_iota(jnp.int32, logits.shape, 1)
        m1 = jnp.max(logits, axis=1, keepdims=True)
        i1 = jnp.min(jnp.where(logits == m1, col, N_EXPERTS), axis=1,
                     keepdims=True)
        l2 = jnp.where(col == i1, -jnp.inf, logits)
        m2 = jnp.max(l2, axis=1, keepdims=True)
        i2 = jnp.min(jnp.where(l2 == m2, col, N_EXPERTS), axis=1,
                     keepdims=True)
        b = jnp.exp(m2 - m1)
        wa = 1.0 / (1.0 + b)
        wb = b / (1.0 + b)
        w_scr[...] = jnp.where(col == i1, wa,
                               jnp.where(col == i2, wb, 0.0))

    w1 = w1_ref[0]
    w2 = w2_ref[0]
    b1v = b1_ref[0]
    b2v = b2_ref[0]

    def body(i, _):
        sl = pl.ds(i * TILE_M, TILE_M)
        h = jnp.maximum(
            jnp.dot(x_ref[sl, :], w1,
                    preferred_element_type=jnp.float32) + b1v, 0.0)
        y = jnp.dot(h, w2, preferred_element_type=jnp.float32) + b2v
        wt = w_scr[sl, :]
        col = jax.lax.broadcasted_iota(jnp.int32, wt.shape, 1)
        we = jnp.sum(jnp.where(col == e, wt, 0.0), axis=1, keepdims=True)
        contrib = we * y

        @pl.when(e == 0)
        def _init():
            out_ref[sl, :] = contrib

        @pl.when(e > 0)
        def _acc():
            out_ref[sl, :] = out_ref[sl, :] + contrib

        return 0

    jax.lax.fori_loop(0, TOKENS // TILE_M, body, 0)


@jax.jit
def kernel(x, gate_W, gate_b, W1, b1, W2, b2):
    gb2d = gate_b.reshape(1, N_EXPERTS)
    b1r = b1.reshape(N_EXPERTS, 1, D_HID)
    b2r = b2.reshape(N_EXPERTS, 1, D_OUT)
    return pl.pallas_call(
        _moe_kernel,
        grid=(N_EXPERTS,),
        in_specs=[
            pl.BlockSpec((TOKENS, D_IN), lambda e: (0, 0)),
            pl.BlockSpec((D_IN, N_EXPERTS), lambda e: (0, 0)),
            pl.BlockSpec((1, N_EXPERTS), lambda e: (0, 0)),
            pl.BlockSpec((1, D_IN, D_HID), lambda e: (e, 0, 0)),
            pl.BlockSpec((1, 1, D_HID), lambda e: (e, 0, 0)),
            pl.BlockSpec((1, D_HID, D_OUT), lambda e: (e, 0, 0)),
            pl.BlockSpec((1, 1, D_OUT), lambda e: (e, 0, 0)),
        ],
        out_specs=pl.BlockSpec((TOKENS, D_OUT), lambda e: (0, 0)),
        out_shape=jax.ShapeDtypeStruct((TOKENS, D_OUT), jnp.float32),
        scratch_shapes=[pltpu.VMEM((TOKENS, N_EXPERTS), jnp.float32)],
    )(x, gate_W, gb2d, W1, b1r, W2, b2r)
